# baseline (speedup 1.0000x reference)
import functools

import jax
import jax.numpy as jnp
from jax import lax
from jax.experimental import pallas as pl
from jax.experimental.pallas import tpu as pltpu
from jax.experimental.pallas import tpu_sc as plsc

MIN_ROUTE_LEN = 2
MAX_ROUTE_LEN = 16


def _dense_kernel(dem_ref, tt_ref, nt_ref, hp_ref, trip_out_ref, scalars_ref):
    dem = dem_ref[...]
    tt = tt_ref[...]
    nt = nt_ref[...]
    hp = hp_ref[...]

    hpf = hp.astype(jnp.float32)
    trip_times = tt * hpf
    trip_out_ref[...] = trip_times

    zero = jnp.zeros((), jnp.float32)
    sd = dem * hpf
    ax = (1, 2)
    p_dt = jnp.sum(dem * trip_times, axis=ax)
    p_td = jnp.sum(dem, axis=ax)
    p_sv = jnp.sum(sd, axis=ax)
    p_tr = jnp.sum(dem * nt.astype(jnp.float32), axis=ax)
    p_t0 = jnp.sum(jnp.where(nt == 0, sd, zero), axis=ax)
    p_t1 = jnp.sum(jnp.where(nt == 1, sd, zero), axis=ax)
    p_t2 = jnp.sum(jnp.where(nt == 2, sd, zero), axis=ax)

    vec = jnp.stack([p_dt, p_t0, p_t1, p_t2, p_td - p_t0 - p_t1 - p_t2,
                     p_td, p_td - p_sv, p_tr], axis=1)
    scalars_ref[:, 0, :] = vec


def _routes_sc_kernel(routes_hbm, nrl_hbm, hcr_hbm, dem_hbm, nsv_hbm, oob_hbm,
                      probe_hbm, routes_v, pos_v, counts_v, nrl_v, hcr_v,
                      oob_v, slab_v):
    R, L = routes_v.shape
    c = lax.axis_index("c")
    s = lax.axis_index("s")
    b = s * 2 + c

    pltpu.sync_copy(routes_hbm.at[b], routes_v)
    pltpu.sync_copy(nrl_hbm, nrl_v)
    pltpu.sync_copy(hcr_hbm, hcr_v)

    lanes = lax.iota(jnp.int32, 16)
    accs = [jnp.zeros((16,), jnp.float32) for _ in range(R // 16)]
    oob_acc = jnp.zeros((16,), jnp.float32)
    for r in range(R):
        idx = routes_v[r, :]
        valid = idx > -1
        safe = jnp.where(valid, idx, 0)
        plsc.store_scatter(pos_v, [safe], lanes, mask=valid)
        g = plsc.load_gather(pos_v, [safe], mask=valid)
        first = jnp.logical_and(g == lanes, valid)
        cnt = plsc.all_reduce_population_count(first).astype(jnp.float32)
        rlen = plsc.all_reduce_population_count(valid)
        delta = jnp.maximum(MIN_ROUTE_LEN - rlen, 0)
        delta = jnp.where(rlen == 0, 0, delta)
        delta = delta + jnp.maximum(rlen - MAX_ROUTE_LEN, 0)
        oob_acc = oob_acc + delta.astype(jnp.float32)
        sel = lanes == (r % 16)
        k = r // 16
        accs[k] = jnp.where(sel, cnt, accs[k])
    for k in range(R // 16):
        counts_v[pl.ds(k * 16, 16)] = accs[k]
    pltpu.sync_copy(counts_v, nsv_hbm.at[b])

    bvec = jnp.full((16,), 0, jnp.int32) + b
    nrlb = plsc.load_gather(nrl_v, [bvec])
    hcrb = plsc.load_gather(hcr_v, [bvec])
    oob_v[...] = oob_acc + (nrlb - hcrb) * float(MIN_ROUTE_LEN)
    pltpu.sync_copy(oob_v, oob_hbm.at[b])
    for chunk in range(4):
        pltpu.sync_copy(dem_hbm.at[b, pl.ds(chunk * 128, 128)], slab_v)
        pltpu.sync_copy(slab_v.at[chunk], probe_hbm.at[b, chunk])


@jax.jit
def _run(demand, transit_times, n_transfers, has_path, batch_routes,
         nrl, hcr):
    B, N, _ = demand.shape
    _, R, L = batch_routes.shape
    G = 4
    bs_full = pl.BlockSpec((G, N, N), lambda g: (g, 0, 0))
    trip_times, scalars = pl.pallas_call(
        _dense_kernel,
        grid=(B // G,),
        in_specs=[bs_full, bs_full, bs_full, bs_full],
        out_specs=[bs_full, pl.BlockSpec((G, 1, 8), lambda g: (g, 0, 0))],
        out_shape=[jax.ShapeDtypeStruct((B, N, N), jnp.float32),
                   jax.ShapeDtypeStruct((B, 1, 8), jnp.float32)],
    )(demand, transit_times, n_transfers, has_path)

    mesh = plsc.VectorSubcoreMesh(core_axis_name="c", subcore_axis_name="s",
                                  num_cores=2, num_subcores=16)
    nsv, oob, _probe = pl.kernel(
        _routes_sc_kernel,
        out_type=[jax.ShapeDtypeStruct((B, R), jnp.float32),
                  jax.ShapeDtypeStruct((B, 16), jnp.float32),
                  jax.ShapeDtypeStruct((B, 4, 512), jnp.float32)],
        mesh=mesh,
        scratch_types=[pltpu.VMEM((R, L), jnp.int32),
                       pltpu.VMEM((N,), jnp.int32),
                       pltpu.VMEM((R,), jnp.float32),
                       pltpu.VMEM((B,), jnp.float32),
                       pltpu.VMEM((B,), jnp.float32),
                       pltpu.VMEM((16,), jnp.float32),
                       pltpu.VMEM((128, 512), jnp.float32)],
        compiler_params=pltpu.CompilerParams(needs_layout_passes=False),
    )(batch_routes, nrl, hcr, demand)
    return trip_times, scalars, nsv, oob


def kernel(demand, transit_times, total_route_time, n_routes_left_to_plan,
           n_transfers, has_path, batch_routes, has_current_route,
           n_disconnected):
    B = demand.shape[0]
    R = batch_routes.shape[1]
    hcr = has_current_route.astype(jnp.float32)
    trip_times, scalars, nsv, oob = _run(
        demand, transit_times, n_transfers, has_path, batch_routes,
        n_routes_left_to_plan, hcr)
    sc = scalars.reshape(B, 8)
    total_dmd_time = sc[:, 0]
    trips_at_transfers = sc[:, 1:5]
    total_demand = sc[:, 5]
    unserved_demand = sc[:, 6]
    total_transfers = sc[:, 7]
    n_stops_oob = oob[:, 0]
    n_stops_visited = nsv
    return (total_dmd_time, total_route_time, trips_at_transfers,
            total_demand, unserved_demand, total_transfers, trip_times,
            n_disconnected, n_stops_oob, n_stops_visited)


# speedup vs baseline: 1.2238x; 1.2238x over previous
import functools

import jax
import jax.numpy as jnp
from jax import lax
from jax.experimental import pallas as pl
from jax.experimental.pallas import tpu as pltpu
from jax.experimental.pallas import tpu_sc as plsc

MIN_ROUTE_LEN = 2
MAX_ROUTE_LEN = 16


def _dense_kernel(dem_ref, tt_ref, nt_ref, hp_ref, trip_out_ref, scalars_ref):
    dem = dem_ref[...]
    tt = tt_ref[...]
    nt = nt_ref[...]
    hp = hp_ref[...]

    hpf = (hp != 0).astype(jnp.float32)
    trip_times = tt * hpf
    trip_out_ref[...] = trip_times

    zero = jnp.zeros((), jnp.float32)
    sd = dem * hpf
    ax = (1, 2)
    p_dt = jnp.sum(dem * trip_times, axis=ax)
    p_td = jnp.sum(dem, axis=ax)
    p_sv = jnp.sum(sd, axis=ax)
    p_tr = jnp.sum(dem * nt.astype(jnp.float32), axis=ax)
    p_t0 = jnp.sum(jnp.where(nt == 0, sd, zero), axis=ax)
    p_t1 = jnp.sum(jnp.where(nt == 1, sd, zero), axis=ax)
    p_t2 = jnp.sum(jnp.where(nt == 2, sd, zero), axis=ax)

    vec = jnp.stack([p_dt, p_t0, p_t1, p_t2, p_td - p_t0 - p_t1 - p_t2,
                     p_td, p_td - p_sv, p_tr], axis=1)
    scalars_ref[:, 0, :] = vec


def _routes_sc_kernel(routes_hbm, nrl_hbm, hcr_hbm, nsv_hbm, oob_hbm,
                      routes_v, pos_v, counts_v, nrl_v, hcr_v, oob_v):
    R, L = routes_v.shape
    c = lax.axis_index("c")
    s = lax.axis_index("s")
    b = s * 2 + c

    pltpu.sync_copy(routes_hbm.at[b], routes_v)
    pltpu.sync_copy(nrl_hbm, nrl_v)
    pltpu.sync_copy(hcr_hbm, hcr_v)

    lanes = lax.iota(jnp.int32, 16)
    accs = [jnp.zeros((16,), jnp.float32) for _ in range(R // 16)]
    oob_acc = jnp.zeros((16,), jnp.float32)
    for r in range(R):
        idx = routes_v[r, :]
        valid = idx > -1
        safe = jnp.where(valid, idx, 0)
        plsc.store_scatter(pos_v, [safe], lanes, mask=valid)
        g = plsc.load_gather(pos_v, [safe], mask=valid)
        first = jnp.logical_and(g == lanes, valid)
        cnt = plsc.all_reduce_population_count(first).astype(jnp.float32)
        rlen = plsc.all_reduce_population_count(valid)
        delta = jnp.maximum(MIN_ROUTE_LEN - rlen, 0)
        delta = jnp.where(rlen == 0, 0, delta)
        delta = delta + jnp.maximum(rlen - MAX_ROUTE_LEN, 0)
        oob_acc = oob_acc + delta.astype(jnp.float32)
        sel = lanes == (r % 16)
        k = r // 16
        accs[k] = jnp.where(sel, cnt, accs[k])
    for k in range(R // 16):
        counts_v[pl.ds(k * 16, 16)] = accs[k]
    pltpu.sync_copy(counts_v, nsv_hbm.at[b])

    bvec = jnp.full((16,), 0, jnp.int32) + b
    nrlb = plsc.load_gather(nrl_v, [bvec])
    hcrb = plsc.load_gather(hcr_v, [bvec])
    oob_v[...] = oob_acc + (nrlb - hcrb) * float(MIN_ROUTE_LEN)
    pltpu.sync_copy(oob_v, oob_hbm.at[b])


@jax.jit
def _run(demand, transit_times, n_transfers, has_path, batch_routes,
         nrl, hcr):
    B, N, _ = demand.shape
    _, R, L = batch_routes.shape
    hp8 = has_path.view(jnp.int8)
    G = 4
    bs_full = pl.BlockSpec((G, N, N), lambda g: (g, 0, 0))
    trip_times, scalars = pl.pallas_call(
        _dense_kernel,
        grid=(B // G,),
        in_specs=[bs_full, bs_full, bs_full, bs_full],
        out_specs=[bs_full, pl.BlockSpec((G, 1, 8), lambda g: (g, 0, 0))],
        out_shape=[jax.ShapeDtypeStruct((B, N, N), jnp.float32),
                   jax.ShapeDtypeStruct((B, 1, 8), jnp.float32)],
    )(demand, transit_times, n_transfers, hp8)

    mesh = plsc.VectorSubcoreMesh(core_axis_name="c", subcore_axis_name="s",
                                  num_cores=2, num_subcores=16)
    nsv, oob = pl.kernel(
        _routes_sc_kernel,
        out_type=[jax.ShapeDtypeStruct((B, R), jnp.float32),
                  jax.ShapeDtypeStruct((B, 16), jnp.float32)],
        mesh=mesh,
        scratch_types=[pltpu.VMEM((R, L), jnp.int32),
                       pltpu.VMEM((N,), jnp.int32),
                       pltpu.VMEM((R,), jnp.float32),
                       pltpu.VMEM((B,), jnp.float32),
                       pltpu.VMEM((B,), jnp.float32),
                       pltpu.VMEM((16,), jnp.float32)],
        compiler_params=pltpu.CompilerParams(needs_layout_passes=False),
    )(batch_routes, nrl, hcr)
    return trip_times, scalars, nsv, oob


def kernel(demand, transit_times, total_route_time, n_routes_left_to_plan,
           n_transfers, has_path, batch_routes, has_current_route,
           n_disconnected):
    B = demand.shape[0]
    R = batch_routes.shape[1]
    hcr = has_current_route.astype(jnp.float32)
    trip_times, scalars, nsv, oob = _run(
        demand, transit_times, n_transfers, has_path, batch_routes,
        n_routes_left_to_plan, hcr)
    sc = scalars.reshape(B, 8)
    total_dmd_time = sc[:, 0]
    trips_at_transfers = sc[:, 1:5]
    total_demand = sc[:, 5]
    unserved_demand = sc[:, 6]
    total_transfers = sc[:, 7]
    n_stops_oob = oob[:, 0]
    n_stops_visited = nsv
    return (total_dmd_time, total_route_time, trips_at_transfers,
            total_demand, unserved_demand, total_transfers, trip_times,
            n_disconnected, n_stops_oob, n_stops_visited)
